# Initial kernel scaffold; baseline (speedup 1.0000x reference)
import jax
import jax.numpy as jnp
from jax.experimental import pallas as pl


def kernel(x, params):
    raise NotImplementedError("write your pallas kernel here")


# speedup vs baseline: 5.7021x; 5.7021x over previous
import functools

import jax
import jax.numpy as jnp
from jax import lax
from jax.experimental import pallas as pl
from jax.experimental.pallas import tpu as pltpu

_INTERPRET = False

NNEI = 16
TDIM = 128
D_STATE = 16
_BN_S = 1.0 / (1.0 + 1e-5) ** 0.5
_INV_SQRT_D = 1.0 / TDIM ** 0.5
_F32 = jnp.float32


def _pc(body, grid, in_specs, out_specs, out_shape, scratch_shapes=()):
    return pl.pallas_call(
        body, grid=grid, in_specs=in_specs, out_specs=out_specs,
        out_shape=out_shape, scratch_shapes=list(scratch_shapes),
        interpret=_INTERPRET)


def _fullspec(shape, ngrid):
    nd = len(shape)
    return pl.BlockSpec(shape, lambda *_: (0,) * nd)


def _dot(a, b):
    return jnp.dot(a, b, preferred_element_type=_F32)


def _topk_setup(xq, xyzT, n):
    qx = xq[:, 0:1]
    qy = xq[:, 1:2]
    qz = xq[:, 2:3]
    sx = xyzT[0:1, :]
    sy = xyzT[1:2, :]
    sz = xyzT[2:3, :]
    return (qx - sx) ** 2 + (qy - sy) ** 2 + (qz - sz) ** 2


def _extract_min(d, iota, n):
    m = jnp.min(d, axis=1, keepdims=True)
    ij = jnp.min(jnp.where(d <= m, iota, n), axis=1, keepdims=True)
    return ij


def _embed(x, p):
    B, N, dp = x.shape
    w1T = p['fc1_w1'].T
    w2T = p['fc1_w2'].T
    b1 = p['fc1_b1'][None, :]
    b2 = p['fc1_b2'][None, :]

    def body(x_ref, w1_ref, b1_ref, w2_ref, b2_ref, o_ref):
        h = jnp.maximum(_dot(x_ref[0], w1_ref[...]) + b1_ref[...], 0.0)
        o_ref[0] = _dot(h, w2_ref[...]) + b2_ref[...]

    return _pc(
        body, (B,),
        [pl.BlockSpec((1, N, dp), lambda b: (b, 0, 0)),
         _fullspec(w1T.shape, 1), _fullspec(b1.shape, 1),
         _fullspec(w2T.shape, 1), _fullspec(b2.shape, 1)],
        pl.BlockSpec((1, N, 32), lambda b: (b, 0, 0)),
        jax.ShapeDtypeStruct((B, N, 32), _F32),
    )(x, w1T, b1, w2T, b2)


def _tb_pre(t, feats):
    B, N, dp = feats.shape
    fc1T = t['fc1_w'].T
    fc1b = t['fc1_b'][None, :]
    wqT = t['wq'].T
    wkT = t['wk'].T
    wvT = t['wv'].T

    def body(f_ref, fc1T_ref, fc1b_ref, wq_ref, wk_ref, wv_ref,
             q_ref, k_ref, v_ref):
        xm = _dot(f_ref[0], fc1T_ref[...]) + fc1b_ref[...]
        q_ref[0] = _dot(xm, wq_ref[...])
        k_ref[0] = _dot(xm, wk_ref[...])
        v_ref[0] = _dot(xm, wv_ref[...])

    osd = jax.ShapeDtypeStruct((B, N, TDIM), _F32)
    ospec = pl.BlockSpec((1, N, TDIM), lambda b: (b, 0, 0))
    return _pc(
        body, (B,),
        [pl.BlockSpec((1, N, dp), lambda b: (b, 0, 0)),
         _fullspec(fc1T.shape, 1), _fullspec(fc1b.shape, 1),
         _fullspec(wqT.shape, 1), _fullspec(wkT.shape, 1),
         _fullspec(wvT.shape, 1)],
        [ospec, ospec, ospec], [osd, osd, osd],
    )(feats, fc1T, fc1b, wqT, wkT, wvT)


def _tb_main_body(k, n, c, dp, xq_ref, xyzT_ref, f_ref, q_ref, src_ref,
                  fd1_ref, fd1b_ref, fd2_ref, fd2b_ref,
                  fg1_ref, fg1b_ref, fg2_ref, fg2b_ref,
                  fc2_ref, fc2b_ref, o_ref):
    xq = xq_ref[0]
    d = _topk_setup(xq, xyzT_ref[0], n)
    iota = lax.broadcasted_iota(jnp.int32, (c, n), 1)
    src = src_ref[0]
    q = q_ref[0]
    logits = []
    wvals = []
    for _ in range(k):
        ij = _extract_min(d, iota, n)
        d = jnp.where(iota == ij, jnp.float32(jnp.inf), d)
        oh = (iota == ij).astype(_F32)
        g = _dot(oh, src)
        kk = g[:, 0:TDIM]
        vv = g[:, TDIM:2 * TDIM]
        nx = g[:, 2 * TDIM:2 * TDIM + 3]
        rel = xq - nx
        pos = _dot(jnp.maximum(_dot(rel, fd1_ref[...]) + fd1b_ref[...], 0.0),
                   fd2_ref[...]) + fd2b_ref[...]
        gg = q - kk + pos
        a = _dot(jnp.maximum(_dot(gg, fg1_ref[...]) + fg1b_ref[...], 0.0),
                 fg2_ref[...]) + fg2b_ref[...]
        logits.append(a * _INV_SQRT_D)
        wvals.append(vv + pos)
    mx = functools.reduce(jnp.maximum, logits)
    den = None
    acc = None
    for l, w in zip(logits, wvals):
        e = jnp.exp(l - mx)
        den = e if den is None else den + e
        ew = e * w
        acc = ew if acc is None else acc + ew
    res = acc / den
    o_ref[0] = _dot(res, fc2_ref[...]) + fc2b_ref[...] + f_ref[0]


def _tb(t, xyz, xyzT, feats):
    B, N, dp = feats.shape
    q, kmat, vmat = _tb_pre(t, feats)
    src = jnp.concatenate([kmat, vmat, xyz], axis=-1)
    k = min(NNEI, N)
    C = 512 if N > 512 else N
    nch = N // C
    fd1T = t['fd_w1'].T
    fd1b = t['fd_b1'][None, :]
    fd2T = t['fd_w2'].T
    fd2b = t['fd_b2'][None, :]
    fg1T = t['fg_w1'].T
    fg1b = t['fg_b1'][None, :]
    fg2T = t['fg_w2'].T
    fg2b = t['fg_b2'][None, :]
    fc2T = t['fc2_w'].T
    fc2b = t['fc2_b'][None, :]
    body = functools.partial(_tb_main_body, k, N, C, dp)
    wspecs = [_fullspec(w.shape, 2) for w in
              (fd1T, fd1b, fd2T, fd2b, fg1T, fg1b, fg2T, fg2b, fc2T, fc2b)]
    return _pc(
        body, (B, nch),
        [pl.BlockSpec((1, C, 3), lambda b, cc: (b, cc, 0)),
         pl.BlockSpec((1, 3, N), lambda b, cc: (b, 0, 0)),
         pl.BlockSpec((1, C, dp), lambda b, cc: (b, cc, 0)),
         pl.BlockSpec((1, C, TDIM), lambda b, cc: (b, cc, 0)),
         pl.BlockSpec((1, N, 2 * TDIM + 3), lambda b, cc: (b, 0, 0))]
        + wspecs,
        pl.BlockSpec((1, C, dp), lambda b, cc: (b, cc, 0)),
        jax.ShapeDtypeStruct((B, N, dp), _F32),
    )(xyz, xyzT, feats, q, src, fd1T, fd1b, fd2T, fd2b,
      fg1T, fg1b, fg2T, fg2b, fc2T, fc2b)


def _fps(xyzT, npoint):
    B, _, N = xyzT.shape

    def body(t_ref, o_ref):
        t = t_ref[0]
        x0 = t[0:1, :]
        x1 = t[1:2, :]
        x2 = t[2:3, :]
        iota = lax.broadcasted_iota(jnp.int32, (1, N), 1)

        def step(j, carry):
            dmin, ij = carry
            oh = (iota == ij).astype(_F32)
            cx = jnp.sum(x0 * oh, axis=1, keepdims=True)
            cy = jnp.sum(x1 * oh, axis=1, keepdims=True)
            cz = jnp.sum(x2 * oh, axis=1, keepdims=True)
            o_ref[0, pl.ds(j, 1), :] = jnp.concatenate([cx, cy, cz], axis=1)
            dn = (x0 - cx) ** 2 + (x1 - cy) ** 2 + (x2 - cz) ** 2
            dmin = jnp.minimum(dmin, dn)
            m = jnp.max(dmin, axis=1, keepdims=True)
            ij2 = jnp.min(jnp.where(dmin >= m, iota, N), axis=1,
                          keepdims=True)
            return dmin, ij2

        init = (jnp.full((1, N), 1e10, _F32), jnp.zeros((1, 1), jnp.int32))
        lax.fori_loop(0, npoint, step, init)

    return _pc(
        body, (B,),
        [pl.BlockSpec((1, 3, N), lambda b: (b, 0, 0))],
        pl.BlockSpec((1, npoint, 3), lambda b: (b, 0, 0)),
        jax.ShapeDtypeStruct((B, npoint, 3), _F32),
    )(xyzT)


def _sa_main_body(k, n, npq, ch_in, xq_ref, xyzT_ref, src_ref,
                  c0a_ref, c0b_ref, c0bias_ref, bn0w_ref, bn0b_ref,
                  c1_ref, c1bias_ref, bn1w_ref, bn1b_ref, o_ref):
    xq = xq_ref[0]
    d = _topk_setup(xq, xyzT_ref[0], n)
    iota = lax.broadcasted_iota(jnp.int32, (npq, n), 1)
    src = src_ref[0]
    hmax = None
    for _ in range(k):
        ij = _extract_min(d, iota, n)
        d = jnp.where(iota == ij, jnp.float32(jnp.inf), d)
        oh = (iota == ij).astype(_F32)
        g = _dot(oh, src)
        fp = g[:, 0:ch_in]
        xp = g[:, ch_in:ch_in + 3]
        norm = xp - xq
        h = _dot(fp, c0b_ref[...]) + _dot(norm, c0a_ref[...]) + c0bias_ref[...]
        h = jnp.maximum(h * _BN_S * bn0w_ref[...] + bn0b_ref[...], 0.0)
        h = _dot(h, c1_ref[...]) + c1bias_ref[...]
        h = jnp.maximum(h * _BN_S * bn1w_ref[...] + bn1b_ref[...], 0.0)
        hmax = h if hmax is None else jnp.maximum(hmax, h)
    o_ref[0] = hmax


def _sa(td, xyzT, xyz, pts, nxyz):
    B, N, ch_in = pts.shape
    npq = nxyz.shape[1]
    ch = td['c0_w'].shape[0]
    src = jnp.concatenate([pts, xyz], axis=-1)
    c0aT = td['c0_w'][:, :3].T
    c0bT = td['c0_w'][:, 3:].T
    c0b = td['c0_b'][None, :]
    bn0w = td['bn0_w'][None, :]
    bn0b = td['bn0_b'][None, :]
    c1T = td['c1_w'].T
    c1b = td['c1_b'][None, :]
    bn1w = td['bn1_w'][None, :]
    bn1b = td['bn1_b'][None, :]
    k = min(NNEI, N)
    body = functools.partial(_sa_main_body, k, N, npq, ch_in)
    wspecs = [_fullspec(w.shape, 1) for w in
              (c0aT, c0bT, c0b, bn0w, bn0b, c1T, c1b, bn1w, bn1b)]
    return _pc(
        body, (B,),
        [pl.BlockSpec((1, npq, 3), lambda b: (b, 0, 0)),
         pl.BlockSpec((1, 3, N), lambda b: (b, 0, 0)),
         pl.BlockSpec((1, N, ch_in + 3), lambda b: (b, 0, 0))]
        + wspecs,
        pl.BlockSpec((1, npq, ch), lambda b: (b, 0, 0)),
        jax.ShapeDtypeStruct((B, npq, ch), _F32),
    )(nxyz, xyzT, src, c0aT, c0bT, c0b, bn0w, bn0b, c1T, c1b, bn1w, bn1b)


def _scan(ss, pts):
    B, N, ch = pts.shape
    A = -jnp.exp(ss['A'])
    A4 = jnp.tile(A, (1, B))
    eye = jnp.eye(B, dtype=_F32)
    Bblk = jnp.kron(eye, ss['B'])
    Cblk = jnp.kron(eye, ss['C'])
    D4 = jnp.tile(ss['D'][None, :], (1, B))
    xf = jnp.transpose(pts, (1, 0, 2)).reshape(N, B * ch)
    BE = B * D_STATE

    def body(xf_ref, Bblk_ref, Cblk_ref, A4_ref, D4_ref, o_ref,
             u_ref, h_ref):
        u_ref[...] = _dot(xf_ref[...], Bblk_ref[...])
        a4 = A4_ref[...]

        def step(j, h):
            u = u_ref[pl.ds(j, 1), :]
            h = h * jnp.exp(jax.nn.softplus(u) * a4) + u
            h_ref[pl.ds(j, 1), :] = h
            return h

        lax.fori_loop(0, N, step, jnp.zeros((1, BE), _F32))
        o_ref[...] = (_dot(h_ref[...], Cblk_ref[...])
                      + xf_ref[...] * D4_ref[...])

    out = _pc(
        body, (1,),
        [_fullspec(xf.shape, 1), _fullspec(Bblk.shape, 1),
         _fullspec(Cblk.shape, 1), _fullspec(A4.shape, 1),
         _fullspec(D4.shape, 1)],
        _fullspec((N, B * ch), 1),
        jax.ShapeDtypeStruct((N, B * ch), _F32),
        scratch_shapes=[pltpu.VMEM((N, BE), _F32),
                        pltpu.VMEM((N, BE), _F32)],
    )(xf, Bblk, Cblk, A4, D4)
    return jnp.transpose(out.reshape(N, B, ch), (1, 0, 2))


def kernel(x, params):
    B, N0, _ = x.shape
    xyz = x[..., :3]
    xyzT = jnp.swapaxes(xyz, 1, 2)
    feats = _embed(x, params)
    pts = _tb(params['tb0'], xyz, xyzT, feats)
    for i in range(4):
        npoint = N0 // 4 ** (i + 1)
        nxyz = _fps(xyzT, npoint)
        pts = _sa(params['td%d' % i], xyzT, xyz, pts, nxyz)
        xyz = nxyz
        xyzT = jnp.swapaxes(xyz, 1, 2)
        pts = _tb(params['tb%d' % (i + 1)], xyz, xyzT, pts)
        pts = _scan(params['ss%d' % i], pts)
    return pts
